# Initial kernel scaffold; baseline (speedup 1.0000x reference)
import functools

import jax
import jax.numpy as jnp
from jax import lax
from jax.experimental import pallas as pl
from jax.experimental.pallas import tpu as pltpu
from jax.experimental.pallas import tpu_sc as plsc

NUM_STOCKS = 1000000
EMBED_DIM = 64
BATCH = 16384
SEQ_LEN = 50
B_TOTAL = BATCH * SEQ_LEN

NC = 2
NS = 16
NW = NC * NS

IDX_W = 128
KROWS = 8
CHUNK = KROWS * IDX_W
ROWS_TOTAL = B_TOTAL // IDX_W
ROWS_PER_W = ROWS_TOTAL // NW
NCHUNK = ROWS_PER_W // KROWS


def _gather_kernel(ids2d, table):
    mesh = plsc.VectorSubcoreMesh(core_axis_name="c", subcore_axis_name="s")

    @functools.partial(
        pl.kernel,
        mesh=mesh,
        out_type=jax.ShapeDtypeStruct((B_TOTAL, EMBED_DIM), jnp.float32),
        scratch_types=[
            pltpu.VMEM((KROWS, IDX_W), jnp.int32),
            pltpu.VMEM((CHUNK, EMBED_DIM), jnp.float32),
            pltpu.SemaphoreType.DMA,
        ],
    )
    def k(ids_hbm, table_hbm, out_hbm, idx_v, rows_v, sem):
        wid = lax.axis_index("s") * NC + lax.axis_index("c")
        row_base = wid * ROWS_PER_W

        @pl.loop(0, NCHUNK)
        def chunk_loop(i):
            row_off = row_base + i * KROWS
            pltpu.sync_copy(ids_hbm.at[pl.ds(row_off, KROWS)], idx_v)
            for j in range(KROWS):
                pltpu.async_copy(
                    table_hbm.at[idx_v.at[j]],
                    rows_v.at[pl.ds(j * IDX_W, IDX_W)],
                    sem,
                )
            for j in range(KROWS):
                pltpu.make_async_copy(
                    table_hbm.at[idx_v.at[j]],
                    rows_v.at[pl.ds(j * IDX_W, IDX_W)],
                    sem,
                ).wait()
            pltpu.sync_copy(
                rows_v, out_hbm.at[pl.ds(row_off * IDX_W, CHUNK)]
            )

    return k(ids2d, table)


def kernel(stock_ids, table):
    ids2d = stock_ids.reshape(ROWS_TOTAL, IDX_W).astype(jnp.int32)
    out = _gather_kernel(ids2d, table)
    return out.reshape(BATCH, SEQ_LEN, EMBED_DIM)


# speedup vs baseline: 1.8440x; 1.8440x over previous
import functools

import jax
import jax.numpy as jnp
from jax import lax
from jax.experimental import pallas as pl
from jax.experimental.pallas import tpu as pltpu
from jax.experimental.pallas import tpu_sc as plsc

NUM_STOCKS = 1000000
EMBED_DIM = 64
BATCH = 16384
SEQ_LEN = 50
B_TOTAL = BATCH * SEQ_LEN

NC = 2
NS = 16
NW = NC * NS

IDX_W = 128
KROWS = 8
CHUNK = KROWS * IDX_W
ROWS_TOTAL = B_TOTAL // IDX_W
ROWS_PER_W = ROWS_TOTAL // NW
NCHUNK = ROWS_PER_W // KROWS


def _gather_kernel(ids2d, table):
    mesh = plsc.VectorSubcoreMesh(core_axis_name="c", subcore_axis_name="s")

    @functools.partial(
        pl.kernel,
        mesh=mesh,
        out_type=jax.ShapeDtypeStruct((B_TOTAL, EMBED_DIM), jnp.float32),
        scratch_types=[
            pltpu.VMEM((KROWS, IDX_W), jnp.int32),
            pltpu.VMEM((CHUNK, EMBED_DIM), jnp.float32),
            pltpu.SemaphoreType.DMA,
        ],
        compiler_params=pltpu.CompilerParams(use_tc_tiling_on_sc=False),
    )
    def k(ids_hbm, table_hbm, out_hbm, idx_v, rows_v, sem):
        wid = lax.axis_index("s") * NC + lax.axis_index("c")
        row_base = wid * ROWS_PER_W

        @pl.loop(0, NCHUNK)
        def chunk_loop(i):
            row_off = row_base + i * KROWS
            pltpu.sync_copy(ids_hbm.at[pl.ds(row_off, KROWS)], idx_v)
            for j in range(KROWS):
                pltpu.async_copy(
                    table_hbm.at[idx_v.at[j]],
                    rows_v.at[pl.ds(j * IDX_W, IDX_W)],
                    sem,
                )
            for j in range(KROWS):
                pltpu.make_async_copy(
                    table_hbm.at[idx_v.at[j]],
                    rows_v.at[pl.ds(j * IDX_W, IDX_W)],
                    sem,
                ).wait()
            pltpu.sync_copy(
                rows_v, out_hbm.at[pl.ds(row_off * IDX_W, CHUNK)]
            )

    return k(ids2d, table)


def kernel(stock_ids, table):
    ids2d = stock_ids.reshape(ROWS_TOTAL, IDX_W).astype(jnp.int32)
    out = _gather_kernel(ids2d, table)
    return out.reshape(BATCH, SEQ_LEN, EMBED_DIM)


# speedup vs baseline: 1.8568x; 1.0070x over previous
import functools

import jax
import jax.numpy as jnp
from jax import lax
from jax.experimental import pallas as pl
from jax.experimental.pallas import tpu as pltpu
from jax.experimental.pallas import tpu_sc as plsc

NUM_STOCKS = 1000000
EMBED_DIM = 64
BATCH = 16384
SEQ_LEN = 50
B_TOTAL = BATCH * SEQ_LEN

NC = 2
NS = 16
NW = NC * NS

IDX_W = 128
KROWS = 4
CHUNK = KROWS * IDX_W
ROWS_TOTAL = B_TOTAL // IDX_W
ROWS_PER_W = ROWS_TOTAL // NW
NCHUNK = ROWS_PER_W // KROWS
NBUF = 2


def _gather_kernel(ids2d, table):
    mesh = plsc.VectorSubcoreMesh(core_axis_name="c", subcore_axis_name="s")

    @functools.partial(
        pl.kernel,
        mesh=mesh,
        out_type=jax.ShapeDtypeStruct((B_TOTAL, EMBED_DIM), jnp.float32),
        scratch_types=[
            pltpu.VMEM((NBUF, KROWS, IDX_W), jnp.int32),
            pltpu.VMEM((NBUF, CHUNK, EMBED_DIM), jnp.float32),
            [pltpu.SemaphoreType.DMA] * NBUF,
        ],
        compiler_params=pltpu.CompilerParams(use_tc_tiling_on_sc=False),
    )
    def k(ids_hbm, table_hbm, out_hbm, idx_v, rows_v, sems):
        wid = lax.axis_index("s") * NC + lax.axis_index("c")
        row_base = wid * ROWS_PER_W

        def fire(i, b):
            row_off = row_base + i * KROWS
            pltpu.sync_copy(ids_hbm.at[pl.ds(row_off, KROWS)], idx_v.at[b])
            for j in range(KROWS):
                pltpu.async_copy(
                    table_hbm.at[idx_v.at[b, j]],
                    rows_v.at[b, pl.ds(j * IDX_W, IDX_W)],
                    sems[b],
                )

        def drain(b):
            for j in range(KROWS):
                pltpu.make_async_copy(
                    table_hbm.at[idx_v.at[b, j]],
                    rows_v.at[b, pl.ds(j * IDX_W, IDX_W)],
                    sems[b],
                ).wait()

        for b in range(NBUF):
            fire(b, b)

        @pl.loop(0, NCHUNK, step=NBUF)
        def chunk_loop(i):
            for b in range(NBUF):
                drain(b)
                row_off = row_base + (i + b) * KROWS
                pltpu.sync_copy(
                    rows_v.at[b], out_hbm.at[pl.ds(row_off * IDX_W, CHUNK)]
                )
                @pl.when(i + b + NBUF < NCHUNK)
                def _():
                    fire(i + b + NBUF, b)

    return k(ids2d, table)


def kernel(stock_ids, table):
    ids2d = stock_ids.reshape(ROWS_TOTAL, IDX_W).astype(jnp.int32)
    out = _gather_kernel(ids2d, table)
    return out.reshape(BATCH, SEQ_LEN, EMBED_DIM)
